# baseline (speedup 1.0000x reference)
import functools

import jax
import jax.numpy as jnp
import numpy as np
from jax import lax
from jax.experimental import pallas as pl
from jax.experimental.pallas import tpu as pltpu
from jax.experimental.pallas import tpu_sc as plsc

_RECAT_IDX = [0, 1, 2, 3, 4, 5, 6, 7, 8, 6, 7, 9, 6, 7, 10, 6, 7, 11,
              6, 7, 12, 6, 7, 13, 6, 7, 14, 6, 7, 15, 0, 3, 6, 1, 4, 7,
              2, 5, 8, 2, 5, 9, 2, 5, 10, 2, 5, 11, 2, 5, 12, 2, 5, 13,
              2, 5, 14, 2, 5, 15]

_NC = 2
_NS = 16
_NW = _NC * _NS

_B, _N, _S, _D = 4, 16, 2048, 128
_ROW = _S * _D
_CH = 2048
_ROWCH = _ROW // _CH
_NJ = len(_RECAT_IDX)
_NQ = _B * _NJ * _ROWCH
_QPW = _NQ // _NW
_G = 16
_NITER = _QPW // _G


def _make_table() -> np.ndarray:
    q = np.arange(_NQ)
    r, c = q // _ROWCH, q % _ROWCH
    b, j = r // _NJ, r % _NJ
    src_row = b * _N + np.asarray(_RECAT_IDX, np.int64)[j]
    return (src_row * _ROWCH + c).astype(np.int32)


_TBL = _make_table()


def _body(x_hbm, tbl_hbm, out_hbm, idx_v, buf_v, sem):
    c = lax.axis_index("c")
    s = lax.axis_index("s")
    w = s * _NC + c
    pltpu.sync_copy(tbl_hbm.at[pl.ds(w * _QPW, _QPW)], idx_v)

    @pl.loop(0, _NITER)
    def _(k):
        ivec = idx_v[pl.ds(k * _G, _G)]
        pltpu.async_copy(x_hbm.at[ivec], buf_v, sem).wait()
        pltpu.sync_copy(buf_v, out_hbm.at[pl.ds(w * _QPW + k * _G, _G)])


@jax.jit
def kernel(x):
    b, n, s, d = x.shape
    x2 = x.reshape(-1, _CH)
    tbl = jnp.asarray(_TBL)
    mesh = plsc.VectorSubcoreMesh(core_axis_name="c", subcore_axis_name="s")
    out = pl.kernel(
        _body,
        out_type=jax.ShapeDtypeStruct((_NQ, _CH), jnp.float32),
        mesh=mesh,
        scratch_types=[
            pltpu.VMEM((_QPW,), jnp.int32),
            pltpu.VMEM((_G, _CH), jnp.float32),
            pltpu.SemaphoreType.DMA,
        ],
    )(x2, tbl)
    return out.reshape(b, _NJ // 3, 3, s, d)


# speedup vs baseline: 1.0690x; 1.0690x over previous
import functools

import jax
import jax.numpy as jnp
import numpy as np
from jax import lax
from jax.experimental import pallas as pl
from jax.experimental.pallas import tpu as pltpu
from jax.experimental.pallas import tpu_sc as plsc

_RECAT_IDX = [0, 1, 2, 3, 4, 5, 6, 7, 8, 6, 7, 9, 6, 7, 10, 6, 7, 11,
              6, 7, 12, 6, 7, 13, 6, 7, 14, 6, 7, 15, 0, 3, 6, 1, 4, 7,
              2, 5, 8, 2, 5, 9, 2, 5, 10, 2, 5, 11, 2, 5, 12, 2, 5, 13,
              2, 5, 14, 2, 5, 15]

_NC = 2
_NS = 16
_NW = _NC * _NS

_B, _N, _S, _D = 4, 16, 2048, 128
_ROW = _S * _D
_CH = 2048
_ROWCH = _ROW // _CH
_NJ = len(_RECAT_IDX)
_NQ = _B * _NJ * _ROWCH
_QPW = _NQ // _NW
_G = 16
_NITER = _QPW // _G


def _make_table() -> np.ndarray:
    q = np.arange(_NQ)
    r, c = q // _ROWCH, q % _ROWCH
    b, j = r // _NJ, r % _NJ
    src_row = b * _N + np.asarray(_RECAT_IDX, np.int64)[j]
    return (src_row * _ROWCH + c).astype(np.int32)


_TBL = _make_table()


def _body(x_hbm, tbl_hbm, out_hbm, idx_v, buf0, buf1, sg0, sg1, ss0, ss1):
    c = lax.axis_index("c")
    s = lax.axis_index("s")
    w = s * _NC + c
    pltpu.sync_copy(tbl_hbm.at[pl.ds(w * _QPW, _QPW)], idx_v)

    bufs, sgs, sss = (buf0, buf1), (sg0, sg1), (ss0, ss1)

    def ivec(k):
        return idx_v[pl.ds(k * _G, _G)]

    def start_gather(k, b):
        pltpu.async_copy(x_hbm.at[ivec(k)], bufs[b], sgs[b])

    def wait_gather(k, b):
        pltpu.make_async_copy(x_hbm.at[ivec(k)], bufs[b], sgs[b]).wait()

    def start_store(k, b):
        pltpu.async_copy(bufs[b], out_hbm.at[pl.ds(w * _QPW + k * _G, _G)],
                         sss[b])

    def wait_store(b):
        pltpu.make_async_copy(bufs[b], out_hbm.at[pl.ds(0, _G)], sss[b]).wait()

    start_gather(0, 0)
    wait_gather(0, 0)
    start_gather(1, 1)
    start_store(0, 0)

    @pl.loop(1, _NITER // 2)
    def _(t):
        k1 = 2 * t - 1
        wait_gather(k1, 1)
        wait_store(0)
        start_gather(k1 + 1, 0)
        start_store(k1, 1)
        k2 = 2 * t
        wait_gather(k2, 0)
        wait_store(1)
        start_gather(k2 + 1, 1)
        start_store(k2, 0)

    wait_gather(_NITER - 1, 1)
    wait_store(0)
    start_store(_NITER - 1, 1)
    wait_store(1)


@jax.jit
def kernel(x):
    b, n, s, d = x.shape
    x2 = x.reshape(-1, _CH)
    tbl = jnp.asarray(_TBL)
    mesh = plsc.VectorSubcoreMesh(core_axis_name="c", subcore_axis_name="s")
    out = pl.kernel(
        _body,
        out_type=jax.ShapeDtypeStruct((_NQ, _CH), jnp.float32),
        mesh=mesh,
        scratch_types=[
            pltpu.VMEM((_QPW,), jnp.int32),
            pltpu.VMEM((_G, _CH), jnp.float32),
            pltpu.VMEM((_G, _CH), jnp.float32),
            pltpu.SemaphoreType.DMA,
            pltpu.SemaphoreType.DMA,
            pltpu.SemaphoreType.DMA,
            pltpu.SemaphoreType.DMA,
        ],
    )(x2, tbl)
    return out.reshape(b, _NJ // 3, 3, s, d)
